# baseline (speedup 1.0000x reference)
import functools

import jax
import jax.numpy as jnp
from jax import lax
from jax.experimental import pallas as pl
from jax.experimental.pallas import tpu as pltpu
from jax.experimental.pallas import tpu_sc as plsc

_K = 8192
_D = 256
_TN = 1024
_LOSS_SCALE = 1.25 / _D


def _argmin_body(zp_ref, w_ref, idx_ref, loss_ref, wnorm_ref, lanes_ref):
    w = w_ref[...]

    @pl.when(pl.program_id(0) == 0)
    def _():
        ww = w * w
        ones = jnp.ones((1, _D), jnp.float32)
        wnorm_ref[...] = lax.dot_general(
            ones, ww, (((1,), (1,)), ((), ())),
            preferred_element_type=jnp.float32)
        lanes_ref[...] = lax.broadcasted_iota(
            jnp.int32, (1, _K), 1).astype(jnp.float32)

    zt = jnp.transpose(zp_ref[0], (1, 0))
    znorm = jnp.sum(zt * zt, axis=1, keepdims=True)
    zw2 = lax.dot_general(zt * (-2.0), w, (((1,), (1,)), ((), ())),
                          preferred_element_type=jnp.float32)
    d = (znorm + wnorm_ref[...]) + zw2
    m = jnp.min(d, axis=1, keepdims=True)
    first = jnp.min(jnp.where(d == m, lanes_ref[...], float(_K)), axis=1)
    idx_ref[0, 0, :] = first.astype(jnp.int32)
    loss_ref[0, 0, :] = m[:, 0] * _LOSS_SCALE


def _tc_argmin(z3, W):
    n = z3.shape[0] * z3.shape[2]
    return pl.pallas_call(
        _argmin_body,
        grid=(n // _TN,),
        in_specs=[
            pl.BlockSpec((1, _D, _TN), lambda i: (i, 0, 0)),
            pl.BlockSpec((_K, _D), lambda i: (0, 0)),
        ],
        out_specs=[
            pl.BlockSpec((1, 1, _TN), lambda i: (i, 0, 0)),
            pl.BlockSpec((1, 1, _TN), lambda i: (i, 0, 0)),
        ],
        out_shape=[
            jax.ShapeDtypeStruct((n // _TN, 1, _TN), jnp.int32),
            jax.ShapeDtypeStruct((n // _TN, 1, _TN), jnp.float32),
        ],
        scratch_shapes=[pltpu.VMEM((1, _K), jnp.float32),
                        pltpu.VMEM((1, _K), jnp.float32)],
    )(z3, W)


def _sc_gather(W, idx):
    info = plsc.get_sparse_core_info()
    nw = info.num_cores * info.num_subcores
    n = idx.shape[0]
    bpw = n // nw
    mesh = plsc.VectorSubcoreMesh(core_axis_name="c", subcore_axis_name="s")

    @functools.partial(
        pl.kernel, mesh=mesh,
        out_type=jax.ShapeDtypeStruct((n, _D), jnp.float32),
        scratch_types=[
            pltpu.VMEM((bpw,), jnp.int32),
            pltpu.VMEM((bpw, _D), jnp.float32),
            pltpu.SemaphoreType.DMA,
        ],
    )
    def gather_k(table_hbm, idx_hbm, out_hbm, idx_v, rows_v, sem):
        wid = lax.axis_index("s") * info.num_cores + lax.axis_index("c")
        base = wid * bpw
        pltpu.sync_copy(idx_hbm.at[pl.ds(base, bpw)], idx_v)
        pltpu.async_copy(table_hbm.at[idx_v], rows_v, sem).wait()
        pltpu.sync_copy(rows_v, out_hbm.at[pl.ds(base, bpw)])

    return gather_k(W, idx)


def kernel(z, W):
    b, c, h, w = z.shape
    n = b * h * w
    idx3, loss3 = _tc_argmin(z.reshape(b, c, h * w), W)
    q = _sc_gather(W, idx3.reshape(n))
    out = jnp.transpose(q.reshape(b, h * w, c), (0, 2, 1)).reshape(b, c, h, w)
    return out, loss3.reshape(b, h, w)


# speedup vs baseline: 1.0951x; 1.0951x over previous
import functools

import jax
import jax.numpy as jnp
from jax import lax
from jax.experimental import pallas as pl
from jax.experimental.pallas import tpu as pltpu
from jax.experimental.pallas import tpu_sc as plsc

_K = 8192
_D = 256
_TN = 1024
_LOSS_SCALE = 1.25 / _D


def _argmin_body(zp_ref, w_ref, idx_ref, loss_ref, wnorm_ref, lanes_ref):
    w = w_ref[...]

    @pl.when(pl.program_id(0) == 0)
    def _():
        ww = w * w
        ones = jnp.ones((1, _D), jnp.float32)
        wnorm_ref[...] = lax.dot_general(
            ones, ww, (((1,), (1,)), ((), ())),
            preferred_element_type=jnp.float32)
        lanes_ref[...] = lax.broadcasted_iota(
            jnp.int32, (1, _K), 1).astype(jnp.float32)

    zt = zp_ref[...]
    znorm = jnp.sum(zt * zt, axis=1, keepdims=True)
    zw2 = lax.dot_general(zt * (-2.0), w, (((1,), (1,)), ((), ())),
                          preferred_element_type=jnp.float32)
    d = (znorm + wnorm_ref[...]) + zw2
    m = jnp.min(d, axis=1, keepdims=True)
    first = jnp.min(jnp.where(d == m, lanes_ref[...], float(_K)), axis=1)
    idx_ref[0, 0, :] = first.astype(jnp.int32)
    loss_ref[0, 0, :] = m[:, 0] * _LOSS_SCALE


def _tc_argmin(zp, W):
    n = zp.shape[0]
    return pl.pallas_call(
        _argmin_body,
        grid=(n // _TN,),
        in_specs=[
            pl.BlockSpec((_TN, _D), lambda i: (i, 0)),
            pl.BlockSpec((_K, _D), lambda i: (0, 0)),
        ],
        out_specs=[
            pl.BlockSpec((1, 1, _TN), lambda i: (i, 0, 0)),
            pl.BlockSpec((1, 1, _TN), lambda i: (i, 0, 0)),
        ],
        out_shape=[
            jax.ShapeDtypeStruct((n // _TN, 1, _TN), jnp.int32),
            jax.ShapeDtypeStruct((n // _TN, 1, _TN), jnp.float32),
        ],
        scratch_shapes=[pltpu.VMEM((1, _K), jnp.float32),
                        pltpu.VMEM((1, _K), jnp.float32)],
    )(zp, W)


def _sc_gather(W, idx):
    info = plsc.get_sparse_core_info()
    nw = info.num_cores * info.num_subcores
    n = idx.shape[0]
    bpw = n // nw
    mesh = plsc.VectorSubcoreMesh(core_axis_name="c", subcore_axis_name="s")

    @functools.partial(
        pl.kernel, mesh=mesh,
        out_type=jax.ShapeDtypeStruct((n, _D), jnp.float32),
        scratch_types=[
            pltpu.VMEM((bpw,), jnp.int32),
            pltpu.VMEM((bpw, _D), jnp.float32),
            pltpu.SemaphoreType.DMA,
        ],
    )
    def gather_k(table_hbm, idx_hbm, out_hbm, idx_v, rows_v, sem):
        wid = lax.axis_index("s") * info.num_cores + lax.axis_index("c")
        base = wid * bpw
        pltpu.sync_copy(idx_hbm.at[pl.ds(base, bpw)], idx_v)
        pltpu.async_copy(table_hbm.at[idx_v], rows_v, sem).wait()
        pltpu.sync_copy(rows_v, out_hbm.at[pl.ds(base, bpw)])

    return gather_k(W, idx)


def kernel(z, W):
    b, c, h, w = z.shape
    n = b * h * w
    zp = jnp.transpose(z.reshape(b, c, h * w), (0, 2, 1)).reshape(n, c)
    idx3, loss3 = _tc_argmin(zp, W)
    q = _sc_gather(W, idx3.reshape(n))
    out = jnp.transpose(q.reshape(b, h * w, c), (0, 2, 1)).reshape(b, c, h, w)
    return out, loss3.reshape(b, h, w)
